# Initial kernel scaffold; baseline (speedup 1.0000x reference)
import jax
import jax.numpy as jnp
from jax.experimental import pallas as pl


def kernel(x_in, conv1_w, conv1_b, conv2_w, conv2_b, conv3_w, conv3_b, cm1_1_w, cm1_1_b, cm2_1_w, cm2_1_b, cm3_1_w, cm3_1_b, cm1_2_w, cm1_2_b, cm2_2_w, cm2_2_b, cm3_2_w, cm3_2_b):
    raise NotImplementedError("write your pallas kernel here")


# speedup vs baseline: 1.5195x; 1.5195x over previous
import jax
import jax.numpy as jnp
from jax.experimental import pallas as pl
from jax.experimental.pallas import tpu as pltpu

_T = 256
_T2 = 256


def _leaky(x):
    return jnp.where(x >= 0, x, 0.01 * x)


def _sel16(h, i1):
    acc = jnp.zeros((32, h.shape[1]), jnp.float32)
    for e in range(16):
        acc = acc + jnp.where((i1 == e)[None, :], h[e * 32:(e + 1) * 32, :], 0.0)
    return acc


def _k1_body(x_ref, w1, b1, w2, b2, w3, b3, wa1, ba1, wa2, ba2, wa3, ba3,
             o_ref, xl_ref):
    X = x_ref[0]
    s = _leaky(jnp.dot(w1[...], X, preferred_element_type=jnp.float32) + b1[...])
    s = _leaky(jnp.dot(w2[...], s, preferred_element_type=jnp.float32) + b2[...])
    s = jnp.dot(w3[...], s, preferred_element_type=jnp.float32) + b3[...]
    i1 = jnp.argmax(s, axis=0).astype(jnp.int32)
    h = _leaky(jnp.dot(wa1[...], X, preferred_element_type=jnp.float32) + ba1[...])
    y = _sel16(h, i1)
    h = _leaky(jnp.dot(wa2[...], y, preferred_element_type=jnp.float32) + ba2[...])
    y = _sel16(h, i1)
    h = jnp.dot(wa3[...], y, preferred_element_type=jnp.float32) + ba3[...]
    y = _sel16(h, i1)
    i2 = jnp.argmax(y, axis=0).astype(jnp.int32)
    i12 = jnp.clip(i1 * 16 + (i2 - 8), 0, 255)
    o_ref[0, 0, :] = i12
    xl_ref[...] = X.T


def _k3_body(e_ref, x_ref, w1_ref, b1_ref, w2_ref, b2_ref, w3_ref, b3_ref,
             o_ref):
    e = e_ref[pl.program_id(0)]
    Xg = x_ref[0]
    z = _leaky(jnp.dot(Xg, w1_ref[0], preferred_element_type=jnp.float32)
               + b1_ref[0, 0, :])
    z = _leaky(jnp.dot(z, w2_ref[0], preferred_element_type=jnp.float32)
               + b2_ref[0, 0, :])
    z = jnp.dot(z, w3_ref[0], preferred_element_type=jnp.float32) + b3_ref[0, 0, :]
    i3 = jnp.argmax(z, axis=1).astype(jnp.int32)
    o_ref[0, 0, :] = jnp.clip(e * 16 + (i3 - 8), 0, 4095)


def kernel(x_in, conv1_w, conv1_b, conv2_w, conv2_b, conv3_w, conv3_b,
           cm1_1_w, cm1_1_b, cm2_1_w, cm2_1_b, cm3_1_w, cm3_1_b,
           cm1_2_w, cm1_2_b, cm2_2_w, cm2_2_b, cm3_2_w, cm3_2_b):
    B, C, H, W = x_in.shape
    HW = H * W
    N = B * HW
    G = N // _T
    GPB = HW // _T
    xr = x_in.reshape(B, C, HW)

    wa1 = cm1_1_w.transpose(0, 2, 1).reshape(512, C)
    wa2 = cm2_1_w.transpose(0, 2, 1).reshape(512, 32)
    wa3 = cm3_1_w.transpose(0, 2, 1).reshape(512, 32)
    ba1 = cm1_1_b.reshape(512, 1)
    ba2 = cm2_1_b.reshape(512, 1)
    ba3 = cm3_1_b.reshape(512, 1)

    const = lambda i: (0, 0)
    k1_out = pl.pallas_call(
        _k1_body,
        grid=(G,),
        in_specs=[
            pl.BlockSpec((1, C, _T), lambda i: (i // GPB, 0, i % GPB)),
            pl.BlockSpec((32, C), const),
            pl.BlockSpec((32, 1), const),
            pl.BlockSpec((32, 32), const),
            pl.BlockSpec((32, 1), const),
            pl.BlockSpec((16, 32), const),
            pl.BlockSpec((16, 1), const),
            pl.BlockSpec((512, C), const),
            pl.BlockSpec((512, 1), const),
            pl.BlockSpec((512, 32), const),
            pl.BlockSpec((512, 1), const),
            pl.BlockSpec((512, 32), const),
            pl.BlockSpec((512, 1), const),
        ],
        out_specs=[
            pl.BlockSpec((1, 1, _T), lambda i: (i, 0, 0)),
            pl.BlockSpec((_T, C), lambda i: (i, 0)),
        ],
        out_shape=[
            jax.ShapeDtypeStruct((G, 1, _T), jnp.int32),
            jax.ShapeDtypeStruct((N, C), jnp.float32),
        ],
        compiler_params=pltpu.CompilerParams(
            dimension_semantics=("arbitrary",)),
    )(xr, conv1_w, conv1_b.reshape(32, 1), conv2_w, conv2_b.reshape(32, 1),
      conv3_w, conv3_b.reshape(16, 1), wa1, ba1, wa2, ba2, wa3, ba3)

    ids = k1_out[0].reshape(N)
    xl = k1_out[1]

    sid, order = jax.lax.sort_key_val(ids, jnp.arange(N, dtype=jnp.int32))
    gs = jnp.searchsorted(sid, jnp.arange(257, dtype=jnp.int32),
                          side='left').astype(jnp.int32)
    counts = gs[1:] - gs[:-1]
    tiles = (counts + _T2 - 1) // _T2
    pstart = (_T2 * (jnp.cumsum(tiles) - tiles)).astype(jnp.int32)
    Mmax = N // _T2 + 256
    tile_e = (jnp.searchsorted(pstart, jnp.arange(Mmax, dtype=jnp.int32) * _T2,
                               side='right') - 1).astype(jnp.int32)
    slot = jnp.arange(Mmax * _T2, dtype=jnp.int32)
    te_full = jnp.broadcast_to(tile_e[:, None], (Mmax, _T2)).reshape(-1)
    src_pos = jnp.clip(gs[te_full] + (slot - pstart[te_full]), 0, N - 1)
    token_src = order[src_pos]
    xg = xl[token_src].reshape(Mmax, _T2, C)

    out3 = pl.pallas_call(
        _k3_body,
        grid_spec=pltpu.PrefetchScalarGridSpec(
            num_scalar_prefetch=1,
            grid=(Mmax,),
            in_specs=[
                pl.BlockSpec((1, _T2, C), lambda i, E: (i, 0, 0)),
                pl.BlockSpec((1, C, 32), lambda i, E: (E[i], 0, 0)),
                pl.BlockSpec((1, 1, 32), lambda i, E: (E[i], 0, 0)),
                pl.BlockSpec((1, 32, 32), lambda i, E: (E[i], 0, 0)),
                pl.BlockSpec((1, 1, 32), lambda i, E: (E[i], 0, 0)),
                pl.BlockSpec((1, 32, 32), lambda i, E: (E[i], 0, 0)),
                pl.BlockSpec((1, 1, 32), lambda i, E: (E[i], 0, 0)),
            ],
            out_specs=pl.BlockSpec((1, 1, _T2), lambda i, E: (i, 0, 0)),
        ),
        out_shape=jax.ShapeDtypeStruct((Mmax, 1, _T2), jnp.int32),
        compiler_params=pltpu.CompilerParams(
            dimension_semantics=("arbitrary",)),
    )(tile_e, xg, cm1_2_w, cm1_2_b.reshape(256, 1, 32), cm2_2_w,
      cm2_2_b.reshape(256, 1, 32), cm3_2_w, cm3_2_b.reshape(256, 1, 32))

    out3_flat = out3.reshape(Mmax * _T2)
    dst_sorted = pstart[sid] + (jnp.arange(N, dtype=jnp.int32) - gs[sid])
    res_sorted = out3_flat[dst_sorted]
    res = jnp.zeros((N,), jnp.int32).at[order].set(res_sorted,
                                                   unique_indices=True)
    return res.reshape(B, 1, H, W)


# speedup vs baseline: 3.2578x; 2.1440x over previous
import jax
import jax.numpy as jnp
from jax.experimental import pallas as pl
from jax.experimental.pallas import tpu as pltpu

_T = 256
_T2 = 256


def _leaky(x):
    return jnp.where(x >= 0, x, 0.01 * x)


def _sel16(h, i1):
    acc = jnp.zeros((32, h.shape[1]), jnp.float32)
    for e in range(16):
        acc = acc + jnp.where((i1 == e)[None, :], h[e * 32:(e + 1) * 32, :], 0.0)
    return acc


def _k1_body(x_ref, w1, b1, w2, b2, w3, b3, wa1, ba1, wa2, ba2, wa3, ba3,
             u_ref, o_ref, xl_ref, rank_ref, hist_ref):
    X = x_ref[0]
    s = _leaky(jnp.dot(w1[...], X, preferred_element_type=jnp.float32) + b1[...])
    s = _leaky(jnp.dot(w2[...], s, preferred_element_type=jnp.float32) + b2[...])
    s = jnp.dot(w3[...], s, preferred_element_type=jnp.float32) + b3[...]
    i1 = jnp.argmax(s, axis=0).astype(jnp.int32)
    h = _leaky(jnp.dot(wa1[...], X, preferred_element_type=jnp.float32) + ba1[...])
    y = _sel16(h, i1)
    h = _leaky(jnp.dot(wa2[...], y, preferred_element_type=jnp.float32) + ba2[...])
    y = _sel16(h, i1)
    h = jnp.dot(wa3[...], y, preferred_element_type=jnp.float32) + ba3[...]
    y = _sel16(h, i1)
    i2 = jnp.argmax(y, axis=0).astype(jnp.int32)
    i12 = jnp.clip(i1 * 16 + (i2 - 8), 0, 255)
    o_ref[0, 0, :] = i12
    xl_ref[...] = X.T
    T = i12.shape[0]
    cls = jax.lax.broadcasted_iota(jnp.int32, (256, T), 0)
    M = (cls == i12[None, :]).astype(jnp.float32)
    P = jnp.dot(M, u_ref[...], preferred_element_type=jnp.float32)
    rank = jnp.sum(P * M, axis=0) - 1.0
    rank_ref[0, 0, :] = rank.astype(jnp.int32)
    hist_ref[0, :, :] = P[:, T - 1:T].astype(jnp.int32)


def _k3_body(e_ref, x_ref, w1_ref, b1_ref, w2_ref, b2_ref, w3_ref, b3_ref,
             o_ref):
    e = e_ref[pl.program_id(0)]
    Xg = x_ref[0]
    z = _leaky(jnp.dot(Xg, w1_ref[0], preferred_element_type=jnp.float32)
               + b1_ref[0, 0, :])
    z = _leaky(jnp.dot(z, w2_ref[0], preferred_element_type=jnp.float32)
               + b2_ref[0, 0, :])
    z = jnp.dot(z, w3_ref[0], preferred_element_type=jnp.float32) + b3_ref[0, 0, :]
    i3 = jnp.argmax(z, axis=1).astype(jnp.int32)
    o_ref[0, 0, :] = jnp.clip(e * 16 + (i3 - 8), 0, 4095)


def kernel(x_in, conv1_w, conv1_b, conv2_w, conv2_b, conv3_w, conv3_b,
           cm1_1_w, cm1_1_b, cm2_1_w, cm2_1_b, cm3_1_w, cm3_1_b,
           cm1_2_w, cm1_2_b, cm2_2_w, cm2_2_b, cm3_2_w, cm3_2_b):
    B, C, H, W = x_in.shape
    HW = H * W
    N = B * HW
    G = N // _T
    GPB = HW // _T
    xr = x_in.reshape(B, C, HW)

    wa1 = cm1_1_w.transpose(0, 2, 1).reshape(512, C)
    wa2 = cm2_1_w.transpose(0, 2, 1).reshape(512, 32)
    wa3 = cm3_1_w.transpose(0, 2, 1).reshape(512, 32)
    ba1 = cm1_1_b.reshape(512, 1)
    ba2 = cm2_1_b.reshape(512, 1)
    ba3 = cm3_1_b.reshape(512, 1)

    const = lambda i: (0, 0)
    k1_out = pl.pallas_call(
        _k1_body,
        grid=(G,),
        in_specs=[
            pl.BlockSpec((1, C, _T), lambda i: (i // GPB, 0, i % GPB)),
            pl.BlockSpec((32, C), const),
            pl.BlockSpec((32, 1), const),
            pl.BlockSpec((32, 32), const),
            pl.BlockSpec((32, 1), const),
            pl.BlockSpec((16, 32), const),
            pl.BlockSpec((16, 1), const),
            pl.BlockSpec((512, C), const),
            pl.BlockSpec((512, 1), const),
            pl.BlockSpec((512, 32), const),
            pl.BlockSpec((512, 1), const),
            pl.BlockSpec((512, 32), const),
            pl.BlockSpec((512, 1), const),
            pl.BlockSpec((_T, _T), const),
        ],
        out_specs=[
            pl.BlockSpec((1, 1, _T), lambda i: (i, 0, 0)),
            pl.BlockSpec((_T, C), lambda i: (i, 0)),
            pl.BlockSpec((1, 1, _T), lambda i: (i, 0, 0)),
            pl.BlockSpec((1, 256, 1), lambda i: (i, 0, 0)),
        ],
        out_shape=[
            jax.ShapeDtypeStruct((G, 1, _T), jnp.int32),
            jax.ShapeDtypeStruct((N, C), jnp.float32),
            jax.ShapeDtypeStruct((G, 1, _T), jnp.int32),
            jax.ShapeDtypeStruct((G, 256, 1), jnp.int32),
        ],
        compiler_params=pltpu.CompilerParams(
            dimension_semantics=("arbitrary",)),
    )(xr, conv1_w, conv1_b.reshape(32, 1), conv2_w, conv2_b.reshape(32, 1),
      conv3_w, conv3_b.reshape(16, 1), wa1, ba1, wa2, ba2, wa3, ba3,
      jnp.triu(jnp.ones((_T, _T), jnp.float32)))

    ids = k1_out[0].reshape(N)
    xl = k1_out[1]
    ranks = k1_out[2].reshape(N)
    hist = k1_out[3].reshape(G, 256)

    base = jnp.cumsum(hist, axis=0) - hist
    counts = base[-1] + hist[-1]
    tiles = (counts + _T2 - 1) // _T2
    pstart = (_T2 * (jnp.cumsum(tiles) - tiles)).astype(jnp.int32)
    Mmax = N // _T2 + 256
    tile_e = (jnp.searchsorted(pstart, jnp.arange(Mmax, dtype=jnp.int32) * _T2,
                               side='right') - 1).astype(jnp.int32)
    tok = jnp.arange(N, dtype=jnp.int32)
    grank = base.reshape(-1)[(tok // _T) * 256 + ids] + ranks
    dst = pstart[ids] + grank
    token_src = jnp.zeros((Mmax * _T2,), jnp.int32).at[dst].set(
        tok, unique_indices=True)
    xg = xl[token_src].reshape(Mmax, _T2, C)

    out3 = pl.pallas_call(
        _k3_body,
        grid_spec=pltpu.PrefetchScalarGridSpec(
            num_scalar_prefetch=1,
            grid=(Mmax,),
            in_specs=[
                pl.BlockSpec((1, _T2, C), lambda i, E: (i, 0, 0)),
                pl.BlockSpec((1, C, 32), lambda i, E: (E[i], 0, 0)),
                pl.BlockSpec((1, 1, 32), lambda i, E: (E[i], 0, 0)),
                pl.BlockSpec((1, 32, 32), lambda i, E: (E[i], 0, 0)),
                pl.BlockSpec((1, 1, 32), lambda i, E: (E[i], 0, 0)),
                pl.BlockSpec((1, 32, 32), lambda i, E: (E[i], 0, 0)),
                pl.BlockSpec((1, 1, 32), lambda i, E: (E[i], 0, 0)),
            ],
            out_specs=pl.BlockSpec((1, 1, _T2), lambda i, E: (i, 0, 0)),
        ),
        out_shape=jax.ShapeDtypeStruct((Mmax, 1, _T2), jnp.int32),
        compiler_params=pltpu.CompilerParams(
            dimension_semantics=("arbitrary",)),
    )(tile_e, xg, cm1_2_w, cm1_2_b.reshape(256, 1, 32), cm2_2_w,
      cm2_2_b.reshape(256, 1, 32), cm3_2_w, cm3_2_b.reshape(256, 1, 32))

    res = out3.reshape(Mmax * _T2)[dst]
    return res.reshape(B, 1, H, W)


# speedup vs baseline: 5.0204x; 1.5411x over previous
import functools

import jax
import jax.numpy as jnp
from jax import lax
from jax.experimental import pallas as pl
from jax.experimental.pallas import tpu as pltpu
from jax.experimental.pallas import tpu_sc as plsc

_T = 256
_T2 = 256
_NC, _NS = 2, 16
_NW = _NC * _NS
_R = 128


def _sc_scatter_rows(xl, dst, m_tot):
    N, C = xl.shape
    rpw = N // _NW
    iters = rpw // _R
    assert rpw % _R == 0 and N % _NW == 0
    mesh = plsc.VectorSubcoreMesh(core_axis_name="c", subcore_axis_name="s",
                                  num_cores=_NC)

    @functools.partial(
        pl.kernel, mesh=mesh,
        out_type=jax.ShapeDtypeStruct((m_tot, C), jnp.float32),
        scratch_types=[
            pltpu.VMEM((2, _R), jnp.int32),
            pltpu.VMEM((2, _R, C), jnp.float32),
            pltpu.SemaphoreType.DMA,
            pltpu.SemaphoreType.DMA,
        ],
    )
    def k(xl_hbm, dst_hbm, out_hbm, idx_v, rows_v, sem_in, sem_out):
        wid = lax.axis_index("s") * _NC + lax.axis_index("c")
        base = wid * rpw

        def fetch2(it, slot):
            off = base + it * _R
            pltpu.async_copy(
                dst_hbm.at[pl.ds(off, _R)], idx_v.at[slot], sem_in)
            pltpu.async_copy(
                xl_hbm.at[pl.ds(off, _R)], rows_v.at[slot], sem_in)

        fetch2(0, 0)

        @pl.loop(0, iters)
        def body(it):
            slot = lax.rem(it, 2)
            pltpu.make_async_copy(dst_hbm.at[pl.ds(base, _R)],
                                  idx_v.at[slot], sem_in).wait()
            pltpu.make_async_copy(xl_hbm.at[pl.ds(base, _R)],
                                  rows_v.at[slot], sem_in).wait()

            @pl.when(it + 1 < iters)
            def _():
                fetch2(it + 1, 1 - slot)

            pltpu.async_copy(rows_v.at[slot], out_hbm.at[idx_v.at[slot]],
                             sem_out)
            pltpu.make_async_copy(rows_v.at[slot],
                                  out_hbm.at[idx_v.at[slot]], sem_out).wait()

    return k(xl, dst)


def _leaky(x):
    return jnp.where(x >= 0, x, 0.01 * x)


def _sel16(h, i1):
    acc = jnp.zeros((32, h.shape[1]), jnp.float32)
    for e in range(16):
        acc = acc + jnp.where((i1 == e)[None, :], h[e * 32:(e + 1) * 32, :], 0.0)
    return acc


def _k1_body(x_ref, w1, b1, w2, b2, w3, b3, wa1, ba1, wa2, ba2, wa3, ba3,
             u_ref, o_ref, xl_ref, rank_ref, hist_ref):
    X = x_ref[0]
    s = _leaky(jnp.dot(w1[...], X, preferred_element_type=jnp.float32) + b1[...])
    s = _leaky(jnp.dot(w2[...], s, preferred_element_type=jnp.float32) + b2[...])
    s = jnp.dot(w3[...], s, preferred_element_type=jnp.float32) + b3[...]
    i1 = jnp.argmax(s, axis=0).astype(jnp.int32)
    h = _leaky(jnp.dot(wa1[...], X, preferred_element_type=jnp.float32) + ba1[...])
    y = _sel16(h, i1)
    h = _leaky(jnp.dot(wa2[...], y, preferred_element_type=jnp.float32) + ba2[...])
    y = _sel16(h, i1)
    h = jnp.dot(wa3[...], y, preferred_element_type=jnp.float32) + ba3[...]
    y = _sel16(h, i1)
    i2 = jnp.argmax(y, axis=0).astype(jnp.int32)
    i12 = jnp.clip(i1 * 16 + (i2 - 8), 0, 255)
    o_ref[0, 0, :] = i12
    xl_ref[...] = X.T
    T = i12.shape[0]
    cls = jax.lax.broadcasted_iota(jnp.int32, (256, T), 0)
    M = (cls == i12[None, :]).astype(jnp.float32)
    P = jnp.dot(M, u_ref[...], preferred_element_type=jnp.float32)
    rank = jnp.sum(P * M, axis=0) - 1.0
    rank_ref[0, 0, :] = rank.astype(jnp.int32)
    hist_ref[0, :, :] = P[:, T - 1:T].astype(jnp.int32)


def _k3_body(e_ref, x_ref, w1_ref, b1_ref, w2_ref, b2_ref, w3_ref, b3_ref,
             o_ref):
    e = e_ref[pl.program_id(0)]
    Xg = x_ref[0]
    z = _leaky(jnp.dot(Xg, w1_ref[0], preferred_element_type=jnp.float32)
               + b1_ref[0, 0, :])
    z = _leaky(jnp.dot(z, w2_ref[0], preferred_element_type=jnp.float32)
               + b2_ref[0, 0, :])
    z = jnp.dot(z, w3_ref[0], preferred_element_type=jnp.float32) + b3_ref[0, 0, :]
    i3 = jnp.argmax(z, axis=1).astype(jnp.int32)
    o_ref[0, 0, :] = jnp.clip(e * 16 + (i3 - 8), 0, 4095)


def kernel(x_in, conv1_w, conv1_b, conv2_w, conv2_b, conv3_w, conv3_b,
           cm1_1_w, cm1_1_b, cm2_1_w, cm2_1_b, cm3_1_w, cm3_1_b,
           cm1_2_w, cm1_2_b, cm2_2_w, cm2_2_b, cm3_2_w, cm3_2_b):
    B, C, H, W = x_in.shape
    HW = H * W
    N = B * HW
    G = N // _T
    GPB = HW // _T
    xr = x_in.reshape(B, C, HW)

    wa1 = cm1_1_w.transpose(0, 2, 1).reshape(512, C)
    wa2 = cm2_1_w.transpose(0, 2, 1).reshape(512, 32)
    wa3 = cm3_1_w.transpose(0, 2, 1).reshape(512, 32)
    ba1 = cm1_1_b.reshape(512, 1)
    ba2 = cm2_1_b.reshape(512, 1)
    ba3 = cm3_1_b.reshape(512, 1)

    const = lambda i: (0, 0)
    k1_out = pl.pallas_call(
        _k1_body,
        grid=(G,),
        in_specs=[
            pl.BlockSpec((1, C, _T), lambda i: (i // GPB, 0, i % GPB)),
            pl.BlockSpec((32, C), const),
            pl.BlockSpec((32, 1), const),
            pl.BlockSpec((32, 32), const),
            pl.BlockSpec((32, 1), const),
            pl.BlockSpec((16, 32), const),
            pl.BlockSpec((16, 1), const),
            pl.BlockSpec((512, C), const),
            pl.BlockSpec((512, 1), const),
            pl.BlockSpec((512, 32), const),
            pl.BlockSpec((512, 1), const),
            pl.BlockSpec((512, 32), const),
            pl.BlockSpec((512, 1), const),
            pl.BlockSpec((_T, _T), const),
        ],
        out_specs=[
            pl.BlockSpec((1, 1, _T), lambda i: (i, 0, 0)),
            pl.BlockSpec((_T, C), lambda i: (i, 0)),
            pl.BlockSpec((1, 1, _T), lambda i: (i, 0, 0)),
            pl.BlockSpec((1, 256, 1), lambda i: (i, 0, 0)),
        ],
        out_shape=[
            jax.ShapeDtypeStruct((G, 1, _T), jnp.int32),
            jax.ShapeDtypeStruct((N, C), jnp.float32),
            jax.ShapeDtypeStruct((G, 1, _T), jnp.int32),
            jax.ShapeDtypeStruct((G, 256, 1), jnp.int32),
        ],
        compiler_params=pltpu.CompilerParams(
            dimension_semantics=("arbitrary",)),
    )(xr, conv1_w, conv1_b.reshape(32, 1), conv2_w, conv2_b.reshape(32, 1),
      conv3_w, conv3_b.reshape(16, 1), wa1, ba1, wa2, ba2, wa3, ba3,
      jnp.triu(jnp.ones((_T, _T), jnp.float32)))

    ids = k1_out[0].reshape(N)
    xl = k1_out[1]
    ranks = k1_out[2].reshape(N)
    hist = k1_out[3].reshape(G, 256)

    base = jnp.cumsum(hist, axis=0) - hist
    counts = base[-1] + hist[-1]
    tiles = (counts + _T2 - 1) // _T2
    pstart = (_T2 * (jnp.cumsum(tiles) - tiles)).astype(jnp.int32)
    Mmax = N // _T2 + 256
    tile_e = (jnp.searchsorted(pstart, jnp.arange(Mmax, dtype=jnp.int32) * _T2,
                               side='right') - 1).astype(jnp.int32)
    tok = jnp.arange(N, dtype=jnp.int32)
    grank = base.reshape(-1)[(tok // _T) * 256 + ids] + ranks
    dst = pstart[ids] + grank
    xg = _sc_scatter_rows(xl, dst, Mmax * _T2).reshape(Mmax, _T2, C)

    out3 = pl.pallas_call(
        _k3_body,
        grid_spec=pltpu.PrefetchScalarGridSpec(
            num_scalar_prefetch=1,
            grid=(Mmax,),
            in_specs=[
                pl.BlockSpec((1, _T2, C), lambda i, E: (i, 0, 0)),
                pl.BlockSpec((1, C, 32), lambda i, E: (E[i], 0, 0)),
                pl.BlockSpec((1, 1, 32), lambda i, E: (E[i], 0, 0)),
                pl.BlockSpec((1, 32, 32), lambda i, E: (E[i], 0, 0)),
                pl.BlockSpec((1, 1, 32), lambda i, E: (E[i], 0, 0)),
                pl.BlockSpec((1, 32, 32), lambda i, E: (E[i], 0, 0)),
                pl.BlockSpec((1, 1, 32), lambda i, E: (E[i], 0, 0)),
            ],
            out_specs=pl.BlockSpec((1, 1, _T2), lambda i, E: (i, 0, 0)),
        ),
        out_shape=jax.ShapeDtypeStruct((Mmax, 1, _T2), jnp.int32),
        compiler_params=pltpu.CompilerParams(
            dimension_semantics=("arbitrary",)),
    )(tile_e, xg, cm1_2_w, cm1_2_b.reshape(256, 1, 32), cm2_2_w,
      cm2_2_b.reshape(256, 1, 32), cm3_2_w, cm3_2_b.reshape(256, 1, 32))

    res = out3.reshape(Mmax * _T2)[dst]
    return res.reshape(B, 1, H, W)
